# baseline (speedup 1.0000x reference)
import functools

import jax
import jax.numpy as jnp
from jax import lax
from jax.experimental import pallas as pl
from jax.experimental.pallas import tpu as pltpu
from jax.experimental.pallas import tpu_sc as plsc

B, S, H = 1, 2048, 768
E, K, FF = 8, 2, 3072
T = B * S
BM = 128
NBLK = (T * K + E * (BM - 1) + BM - 1) // BM
P_PAD = NBLK * BM
LANES = 128
NW = 32
TPW = T // NW
CHUNK = 256
NEG = -1e30


def _router_body(x_ref, wr_ref, sel_ref, ew_ref, wexp_ref, dest_ref,
                 counts_ref, loss_ref):
    x = x_ref[...]
    logits = jnp.dot(x, wr_ref[...], preferred_element_type=jnp.float32)
    lane = lax.broadcasted_iota(jnp.int32, (T, LANES), 1)
    ok = lane < E
    lg = jnp.where(ok, logits, NEG)
    m = jnp.max(lg, axis=1, keepdims=True)
    p = jnp.exp(lg - m)
    p0 = jnp.max(p, axis=1, keepdims=True)
    e0 = jnp.min(jnp.where(p == p0, lane, LANES), axis=1, keepdims=True)
    pm = jnp.where(lane == e0, -1.0, p)
    pm = jnp.where(ok, pm, -1.0)
    p1 = jnp.max(pm, axis=1, keepdims=True)
    e1 = jnp.min(jnp.where(pm == p1, lane, LANES), axis=1, keepdims=True)
    denom = p0 + p1
    w0 = p0 / denom
    w1 = p1 / denom
    zl = jnp.sum(m + jnp.log(jnp.sum(p, axis=1, keepdims=True))) / T
    oh0 = jnp.where(lane == e0, 1.0, 0.0)
    oh1 = jnp.where(lane == e1, 1.0, 0.0)
    ohsum = oh0 + oh1
    r_iota = lax.broadcasted_iota(jnp.int32, (CHUNK, CHUNK), 0)
    c_iota = lax.broadcasted_iota(jnp.int32, (CHUNK, CHUNK), 1)
    tril = jnp.where(r_iota > c_iota, 1.0, 0.0)
    base = jnp.zeros((1, LANES), jnp.float32)
    cum_parts = []
    for c in range(T // CHUNK):
        oc = lax.slice(ohsum, (c * CHUNK, 0), ((c + 1) * CHUNK, LANES))
        cum_parts.append(jnp.dot(tril, oc, preferred_element_type=jnp.float32)
                         + base)
        base = base + jnp.sum(oc, axis=0, keepdims=True)
    cum = jnp.concatenate(cum_parts, axis=0)
    counts = base
    ci = counts.astype(jnp.int32)
    pad_count = ((ci + (BM - 1)) >> 7) << 7
    l_r = lax.broadcasted_iota(jnp.int32, (LANES, LANES), 0)
    l_c = lax.broadcasted_iota(jnp.int32, (LANES, LANES), 1)
    ltri = jnp.where(l_r < l_c, 1.0, 0.0)
    pad_start = jnp.dot(pad_count.astype(jnp.float32), ltri,
                        preferred_element_type=jnp.float32)
    d0 = jnp.sum(oh0 * (pad_start + cum), axis=1, keepdims=True)
    d1 = jnp.sum(oh1 * (pad_start + cum), axis=1, keepdims=True)
    load = counts / (T * K)
    dev = jnp.where(ok[:1, :], (load - 1.0 / E) ** 2, 0.0)
    aux = jnp.sum(dev) / E
    sel_ref[...] = jnp.concatenate([e0, e1], axis=1)
    ew_ref[...] = jnp.concatenate([w0, w1], axis=1)
    wexp_ref[...] = jnp.concatenate([jnp.broadcast_to(w0, (T, 16)),
                                     jnp.broadcast_to(w1, (T, 16))], axis=1)
    dest_ref[...] = jnp.concatenate([d0.astype(jnp.int32),
                                     d1.astype(jnp.int32)], axis=1)
    counts_ref[...] = counts
    loss_ref[...] = jnp.full((1, 1), 0.0) + 0.01 * aux + 0.001 * zl


def _router(x, wr_pad):
    return pl.pallas_call(
        _router_body,
        out_shape=[
            jax.ShapeDtypeStruct((T, 2), jnp.int32),
            jax.ShapeDtypeStruct((T, 2), jnp.float32),
            jax.ShapeDtypeStruct((T, 32), jnp.float32),
            jax.ShapeDtypeStruct((T, 2), jnp.int32),
            jax.ShapeDtypeStruct((1, LANES), jnp.float32),
            jax.ShapeDtypeStruct((1, 1), jnp.float32),
        ],
    )(x, wr_pad)


def _dispatch_body(x_hbm, destT_hbm, xs_hbm, idx_v, rows_v, sem):
    wid = lax.axis_index("s") * 2 + lax.axis_index("c")
    base = wid * TPW
    pltpu.sync_copy(x_hbm.at[pl.ds(base, TPW)], rows_v)
    pltpu.sync_copy(destT_hbm.at[0, pl.ds(base, TPW)], idx_v)
    pltpu.async_copy(rows_v, xs_hbm.at[idx_v], sem).wait()
    pltpu.sync_copy(destT_hbm.at[1, pl.ds(base, TPW)], idx_v)
    pltpu.async_copy(rows_v, xs_hbm.at[idx_v], sem).wait()


@jax.jit
def _dispatch(x, destT):
    mesh = plsc.VectorSubcoreMesh(core_axis_name="c", subcore_axis_name="s")
    return pl.kernel(
        _dispatch_body,
        mesh=mesh,
        out_type=jax.ShapeDtypeStruct((P_PAD, H), jnp.float32),
        scratch_types=[
            pltpu.VMEM((TPW,), jnp.int32),
            pltpu.VMEM((TPW, H), jnp.float32),
            pltpu.SemaphoreType.DMA,
        ],
    )(x, destT)


def _ffn_body(be_ref, xs_ref, w1_ref, b1_ref, w2_ref, b2_ref, out_ref,
              w1c_ref, w2c_ref, laste_ref):
    i = pl.program_id(0)
    e = be_ref[i]

    @pl.when((i == 0) | (e != laste_ref[0]))
    def _cast():
        w1c_ref[...] = w1_ref[0].astype(jnp.bfloat16)
        w2c_ref[...] = w2_ref[0].astype(jnp.bfloat16)

    laste_ref[0] = e
    xb = xs_ref[...].astype(jnp.bfloat16)
    h1 = jnp.dot(xb, w1c_ref[...], preferred_element_type=jnp.float32)
    h1 = h1 + b1_ref[0]
    hmid = h1 * 0.5 * (1.0 + lax.erf(h1 * 0.7071067811865476))
    y = jnp.dot(hmid.astype(jnp.bfloat16), w2c_ref[...],
                preferred_element_type=jnp.float32)
    out_ref[...] = y + b2_ref[0]


def _ffn(block_expert, xs, W1, b1, W2, b2):
    grid_spec = pltpu.PrefetchScalarGridSpec(
        num_scalar_prefetch=1,
        grid=(NBLK,),
        in_specs=[
            pl.BlockSpec((BM, H), lambda i, be: (i, 0)),
            pl.BlockSpec((1, H, FF), lambda i, be: (be[i], 0, 0)),
            pl.BlockSpec((1, 1, FF), lambda i, be: (be[i], 0, 0)),
            pl.BlockSpec((1, FF, H), lambda i, be: (be[i], 0, 0)),
            pl.BlockSpec((1, 1, H), lambda i, be: (be[i], 0, 0)),
        ],
        out_specs=pl.BlockSpec((BM, H), lambda i, be: (i, 0)),
        scratch_shapes=[
            pltpu.VMEM((H, FF), jnp.bfloat16),
            pltpu.VMEM((FF, H), jnp.bfloat16),
            pltpu.SMEM((1,), jnp.int32),
        ],
    )
    return pl.pallas_call(
        _ffn_body,
        grid_spec=grid_spec,
        out_shape=jax.ShapeDtypeStruct((P_PAD, H), jnp.float32),
    )(block_expert, xs, W1, b1.reshape(E, 1, FF), W2, b2.reshape(E, 1, H))


def _combine_body(ys_hbm, destT_hbm, wexp_hbm, out_hbm,
                  idx_v, w_v, rows0_v, rows1_v, sem):
    wid = lax.axis_index("s") * 2 + lax.axis_index("c")
    base = wid * TPW
    pltpu.sync_copy(destT_hbm.at[0, pl.ds(base, TPW)], idx_v)
    pltpu.async_copy(ys_hbm.at[idx_v], rows0_v, sem).wait()
    pltpu.sync_copy(destT_hbm.at[1, pl.ds(base, TPW)], idx_v)
    pltpu.async_copy(ys_hbm.at[idx_v], rows1_v, sem).wait()
    pltpu.sync_copy(wexp_hbm.at[pl.ds(base, TPW)], w_v)

    def tok(j, _):
        wv0 = w_v[j, pl.ds(0, 16)]
        wv1 = w_v[j, pl.ds(16, 16)]
        for c in range(H // 16):
            sl = pl.ds(c * 16, 16)
            v = rows0_v[j, sl] * wv0 + rows1_v[j, sl] * wv1
            rows0_v[j, sl] = v
        return 0

    lax.fori_loop(0, TPW, tok, 0)
    pltpu.sync_copy(rows0_v, out_hbm.at[pl.ds(base, TPW)])


@jax.jit
def _combine(ys, destT, wexp):
    mesh = plsc.VectorSubcoreMesh(core_axis_name="c", subcore_axis_name="s")
    return pl.kernel(
        _combine_body,
        mesh=mesh,
        out_type=jax.ShapeDtypeStruct((T, H), jnp.float32),
        scratch_types=[
            pltpu.VMEM((TPW,), jnp.int32),
            pltpu.VMEM((TPW, 32), jnp.float32),
            pltpu.VMEM((TPW, H), jnp.float32),
            pltpu.VMEM((TPW, H), jnp.float32),
            pltpu.SemaphoreType.DMA,
        ],
    )(ys, destT, wexp)


def kernel(hidden_states, Wr, W1, b1, W2, b2):
    x = hidden_states.reshape(T, H)
    wr_pad = jnp.zeros((H, LANES), jnp.float32).at[:, :E].set(Wr)
    sel, ew, wexp, dest, counts, loss = _router(x, wr_pad)
    cnt = counts[0, :E].astype(jnp.int32)
    pad_count = ((cnt + (BM - 1)) // BM) * BM
    pad_start = jnp.cumsum(pad_count) - pad_count
    blk = jnp.arange(NBLK, dtype=jnp.int32) * BM
    block_expert = jnp.sum(blk[:, None] >= pad_start[None, :],
                           axis=1).astype(jnp.int32) - 1
    block_expert = jnp.clip(block_expert, 0, E - 1)
    destT = dest.T
    xs = _dispatch(x, destT)
    ys = _ffn(block_expert, xs, W1, b1, W2, b2)
    out = _combine(ys, destT, wexp)
    return (out.reshape(B, S, H), sel, ew, loss[0, 0])


# speedup vs baseline: 1.0965x; 1.0965x over previous
import functools

import jax
import jax.numpy as jnp
from jax import lax
from jax.experimental import pallas as pl
from jax.experimental.pallas import tpu as pltpu
from jax.experimental.pallas import tpu_sc as plsc

B, S, H = 1, 2048, 768
E, K, FF = 8, 2, 3072
T = B * S
BM = 256
BMLOG = BM.bit_length() - 1
NBLK = (T * K + E * (BM - 1) + BM - 1) // BM
P_PAD = NBLK * BM
LANES = 128
NW = 32
TPW = T // NW
CHUNK = 256
NEG = -1e30


def _router_body(x_ref, wr_ref, sel_ref, ew_ref, wexp_ref, dest_ref,
                 counts_ref, loss_ref):
    x = x_ref[...]
    logits = jnp.dot(x, wr_ref[...], preferred_element_type=jnp.float32)
    lane = lax.broadcasted_iota(jnp.int32, (T, LANES), 1)
    ok = lane < E
    lg = jnp.where(ok, logits, NEG)
    m = jnp.max(lg, axis=1, keepdims=True)
    p = jnp.exp(lg - m)
    p0 = jnp.max(p, axis=1, keepdims=True)
    e0 = jnp.min(jnp.where(p == p0, lane, LANES), axis=1, keepdims=True)
    pm = jnp.where(lane == e0, -1.0, p)
    pm = jnp.where(ok, pm, -1.0)
    p1 = jnp.max(pm, axis=1, keepdims=True)
    e1 = jnp.min(jnp.where(pm == p1, lane, LANES), axis=1, keepdims=True)
    denom = p0 + p1
    w0 = p0 / denom
    w1 = p1 / denom
    zl = jnp.sum(m + jnp.log(jnp.sum(p, axis=1, keepdims=True))) / T
    oh0 = jnp.where(lane == e0, 1.0, 0.0)
    oh1 = jnp.where(lane == e1, 1.0, 0.0)
    ohsum = oh0 + oh1
    r_iota = lax.broadcasted_iota(jnp.int32, (CHUNK, CHUNK), 0)
    c_iota = lax.broadcasted_iota(jnp.int32, (CHUNK, CHUNK), 1)
    tril = jnp.where(r_iota > c_iota, 1.0, 0.0)
    base = jnp.zeros((1, LANES), jnp.float32)
    cum_parts = []
    for c in range(T // CHUNK):
        oc = lax.slice(ohsum, (c * CHUNK, 0), ((c + 1) * CHUNK, LANES))
        cum_parts.append(jnp.dot(tril, oc, preferred_element_type=jnp.float32)
                         + base)
        base = base + jnp.sum(oc, axis=0, keepdims=True)
    cum = jnp.concatenate(cum_parts, axis=0)
    counts = base
    ci = counts.astype(jnp.int32)
    pad_count = ((ci + (BM - 1)) >> BMLOG) << BMLOG
    l_r = lax.broadcasted_iota(jnp.int32, (LANES, LANES), 0)
    l_c = lax.broadcasted_iota(jnp.int32, (LANES, LANES), 1)
    ltri = jnp.where(l_r < l_c, 1.0, 0.0)
    pad_start = jnp.dot(pad_count.astype(jnp.float32), ltri,
                        preferred_element_type=jnp.float32)
    d0 = jnp.sum(oh0 * (pad_start + cum), axis=1, keepdims=True)
    d1 = jnp.sum(oh1 * (pad_start + cum), axis=1, keepdims=True)
    load = counts / (T * K)
    dev = jnp.where(ok[:1, :], (load - 1.0 / E) ** 2, 0.0)
    aux = jnp.sum(dev) / E
    sel_ref[...] = jnp.concatenate([e0, e1], axis=1)
    ew_ref[...] = jnp.concatenate([w0, w1], axis=1)
    wexp_ref[...] = jnp.concatenate([jnp.broadcast_to(w0, (T, 16)),
                                     jnp.broadcast_to(w1, (T, 16))], axis=1)
    dest_ref[...] = jnp.concatenate([d0.astype(jnp.int32),
                                     d1.astype(jnp.int32)], axis=1)
    counts_ref[...] = counts
    loss_ref[...] = jnp.full((1, 1), 0.0) + 0.01 * aux + 0.001 * zl


def _router(x, wr_pad):
    return pl.pallas_call(
        _router_body,
        out_shape=[
            jax.ShapeDtypeStruct((T, 2), jnp.int32),
            jax.ShapeDtypeStruct((T, 2), jnp.float32),
            jax.ShapeDtypeStruct((T, 32), jnp.float32),
            jax.ShapeDtypeStruct((T, 2), jnp.int32),
            jax.ShapeDtypeStruct((1, LANES), jnp.float32),
            jax.ShapeDtypeStruct((1, 1), jnp.float32),
        ],
    )(x, wr_pad)


def _dispatch_body(x_hbm, destT_hbm, xs_hbm, idx_v, rows_v, sem):
    wid = lax.axis_index("s") * 2 + lax.axis_index("c")
    base = wid * TPW
    pltpu.sync_copy(x_hbm.at[pl.ds(base, TPW)], rows_v)
    pltpu.sync_copy(destT_hbm.at[0, pl.ds(base, TPW)], idx_v)
    pltpu.async_copy(rows_v, xs_hbm.at[idx_v], sem).wait()
    pltpu.sync_copy(destT_hbm.at[1, pl.ds(base, TPW)], idx_v)
    pltpu.async_copy(rows_v, xs_hbm.at[idx_v], sem).wait()


@jax.jit
def _dispatch(x, destT):
    mesh = plsc.VectorSubcoreMesh(core_axis_name="c", subcore_axis_name="s")
    return pl.kernel(
        _dispatch_body,
        mesh=mesh,
        out_type=jax.ShapeDtypeStruct((P_PAD, H), jnp.float32),
        scratch_types=[
            pltpu.VMEM((TPW,), jnp.int32),
            pltpu.VMEM((TPW, H), jnp.float32),
            pltpu.SemaphoreType.DMA,
        ],
    )(x, destT)


def _ffn_body(be_ref, xs_ref, w1_ref, b1_ref, w2_ref, b2_ref, out_ref):
    xb = xs_ref[...]
    h1 = jnp.dot(xb, w1_ref[0], preferred_element_type=jnp.float32)
    h1 = h1 + b1_ref[0]
    hmid = h1 * 0.5 * (1.0 + lax.erf(h1 * 0.7071067811865476))
    y = jnp.dot(hmid, w2_ref[0], preferred_element_type=jnp.float32)
    out_ref[...] = y + b2_ref[0]


def _ffn(block_expert, xs, W1, b1, W2, b2):
    grid_spec = pltpu.PrefetchScalarGridSpec(
        num_scalar_prefetch=1,
        grid=(NBLK,),
        in_specs=[
            pl.BlockSpec((BM, H), lambda i, be: (i, 0)),
            pl.BlockSpec((1, H, FF), lambda i, be: (be[i], 0, 0)),
            pl.BlockSpec((1, 1, FF), lambda i, be: (be[i], 0, 0)),
            pl.BlockSpec((1, FF, H), lambda i, be: (be[i], 0, 0)),
            pl.BlockSpec((1, 1, H), lambda i, be: (be[i], 0, 0)),
        ],
        out_specs=pl.BlockSpec((BM, H), lambda i, be: (i, 0)),
    )
    return pl.pallas_call(
        _ffn_body,
        grid_spec=grid_spec,
        out_shape=jax.ShapeDtypeStruct((P_PAD, H), jnp.float32),
    )(block_expert, xs, W1, b1.reshape(E, 1, FF), W2, b2.reshape(E, 1, H))


def _combine_body(ys_hbm, destT_hbm, wexp_hbm, out_hbm,
                  idx_v, w_v, rows0_v, rows1_v, sem):
    wid = lax.axis_index("s") * 2 + lax.axis_index("c")
    base = wid * TPW
    pltpu.sync_copy(destT_hbm.at[0, pl.ds(base, TPW)], idx_v)
    pltpu.async_copy(ys_hbm.at[idx_v], rows0_v, sem).wait()
    pltpu.sync_copy(destT_hbm.at[1, pl.ds(base, TPW)], idx_v)
    pltpu.async_copy(ys_hbm.at[idx_v], rows1_v, sem).wait()
    pltpu.sync_copy(wexp_hbm.at[pl.ds(base, TPW)], w_v)

    def tok(j, _):
        wv0 = w_v[j, pl.ds(0, 16)]
        wv1 = w_v[j, pl.ds(16, 16)]
        for c in range(H // 16):
            sl = pl.ds(c * 16, 16)
            v = rows0_v[j, sl] * wv0 + rows1_v[j, sl] * wv1
            rows0_v[j, sl] = v
        return 0

    lax.fori_loop(0, TPW, tok, 0)
    pltpu.sync_copy(rows0_v, out_hbm.at[pl.ds(base, TPW)])


@jax.jit
def _combine(ys, destT, wexp):
    mesh = plsc.VectorSubcoreMesh(core_axis_name="c", subcore_axis_name="s")
    return pl.kernel(
        _combine_body,
        mesh=mesh,
        out_type=jax.ShapeDtypeStruct((T, H), jnp.float32),
        scratch_types=[
            pltpu.VMEM((TPW,), jnp.int32),
            pltpu.VMEM((TPW, 32), jnp.float32),
            pltpu.VMEM((TPW, H), jnp.float32),
            pltpu.VMEM((TPW, H), jnp.float32),
            pltpu.SemaphoreType.DMA,
        ],
    )(ys, destT, wexp)


def kernel(hidden_states, Wr, W1, b1, W2, b2):
    x = hidden_states.reshape(T, H)
    wr_pad = jnp.zeros((H, LANES), jnp.float32).at[:, :E].set(Wr)
    sel, ew, wexp, dest, counts, loss = _router(x, wr_pad)
    cnt = counts[0, :E].astype(jnp.int32)
    pad_count = ((cnt + (BM - 1)) // BM) * BM
    pad_start = jnp.cumsum(pad_count) - pad_count
    blk = jnp.arange(NBLK, dtype=jnp.int32) * BM
    block_expert = jnp.sum(blk[:, None] >= pad_start[None, :],
                           axis=1).astype(jnp.int32) - 1
    block_expert = jnp.clip(block_expert, 0, E - 1)
    destT = dest.T
    xs = _dispatch(x, destT)
    ys = _ffn(block_expert, xs, W1, b1, W2, b2)
    out = _combine(ys, destT, wexp)
    return (out.reshape(B, S, H), sel, ew, loss[0, 0])


# speedup vs baseline: 1.1239x; 1.0250x over previous
import functools

import jax
import jax.numpy as jnp
from jax import lax
from jax.experimental import pallas as pl
from jax.experimental.pallas import tpu as pltpu
from jax.experimental.pallas import tpu_sc as plsc

B, S, H = 1, 2048, 768
E, K, FF = 8, 2, 3072
T = B * S
BM = 128
BMLOG = BM.bit_length() - 1
NBLK = (T * K + E * (BM - 1) + BM - 1) // BM
P_PAD = NBLK * BM
LANES = 128
NW = 32
TPW = T // NW
CHUNK = 256
NEG = -1e30


def _router_body(x_ref, wr_ref, sel_ref, ew_ref, wexp_ref, dest_ref,
                 counts_ref, loss_ref):
    x = x_ref[...]
    logits = jnp.dot(x, wr_ref[...], preferred_element_type=jnp.float32)
    lane = lax.broadcasted_iota(jnp.int32, (T, LANES), 1)
    ok = lane < E
    lg = jnp.where(ok, logits, NEG)
    m = jnp.max(lg, axis=1, keepdims=True)
    p = jnp.exp(lg - m)
    p0 = jnp.max(p, axis=1, keepdims=True)
    e0 = jnp.min(jnp.where(p == p0, lane, LANES), axis=1, keepdims=True)
    pm = jnp.where(lane == e0, -1.0, p)
    pm = jnp.where(ok, pm, -1.0)
    p1 = jnp.max(pm, axis=1, keepdims=True)
    e1 = jnp.min(jnp.where(pm == p1, lane, LANES), axis=1, keepdims=True)
    denom = p0 + p1
    w0 = p0 / denom
    w1 = p1 / denom
    zl = jnp.sum(m + jnp.log(jnp.sum(p, axis=1, keepdims=True))) / T
    oh0 = jnp.where(lane == e0, 1.0, 0.0)
    oh1 = jnp.where(lane == e1, 1.0, 0.0)
    ohsum = oh0 + oh1
    r_iota = lax.broadcasted_iota(jnp.int32, (CHUNK, CHUNK), 0)
    c_iota = lax.broadcasted_iota(jnp.int32, (CHUNK, CHUNK), 1)
    tril = jnp.where(r_iota > c_iota, 1.0, 0.0)
    base = jnp.zeros((1, LANES), jnp.float32)
    cum_parts = []
    for c in range(T // CHUNK):
        oc = lax.slice(ohsum, (c * CHUNK, 0), ((c + 1) * CHUNK, LANES))
        cum_parts.append(jnp.dot(tril, oc, preferred_element_type=jnp.float32)
                         + base)
        base = base + jnp.sum(oc, axis=0, keepdims=True)
    cum = jnp.concatenate(cum_parts, axis=0)
    counts = base
    ci = counts.astype(jnp.int32)
    pad_count = ((ci + (BM - 1)) >> BMLOG) << BMLOG
    l_r = lax.broadcasted_iota(jnp.int32, (LANES, LANES), 0)
    l_c = lax.broadcasted_iota(jnp.int32, (LANES, LANES), 1)
    ltri = jnp.where(l_r < l_c, 1.0, 0.0)
    pad_start = jnp.dot(pad_count.astype(jnp.float32), ltri,
                        preferred_element_type=jnp.float32)
    d0 = jnp.sum(oh0 * (pad_start + cum), axis=1, keepdims=True)
    d1 = jnp.sum(oh1 * (pad_start + cum), axis=1, keepdims=True)
    load = counts / (T * K)
    dev = jnp.where(ok[:1, :], (load - 1.0 / E) ** 2, 0.0)
    aux = jnp.sum(dev) / E
    sel_ref[...] = jnp.concatenate([e0, e1], axis=1)
    ew_ref[...] = jnp.concatenate([w0, w1], axis=1)
    wexp_ref[...] = jnp.concatenate([jnp.broadcast_to(w0, (T, 16)),
                                     jnp.broadcast_to(w1, (T, 16))], axis=1)
    dest_ref[...] = jnp.concatenate([d0.astype(jnp.int32),
                                     d1.astype(jnp.int32)], axis=1)
    counts_ref[...] = counts
    loss_ref[...] = jnp.full((1, 1), 0.0) + 0.01 * aux + 0.001 * zl


def _router(x, wr_pad):
    return pl.pallas_call(
        _router_body,
        out_shape=[
            jax.ShapeDtypeStruct((T, 2), jnp.int32),
            jax.ShapeDtypeStruct((T, 2), jnp.float32),
            jax.ShapeDtypeStruct((T, 32), jnp.float32),
            jax.ShapeDtypeStruct((T, 2), jnp.int32),
            jax.ShapeDtypeStruct((1, LANES), jnp.float32),
            jax.ShapeDtypeStruct((1, 1), jnp.float32),
        ],
    )(x, wr_pad)


def _dispatch_body(x_hbm, destT_hbm, xs_hbm, idx_v, rows_v, sem):
    wid = lax.axis_index("s") * 2 + lax.axis_index("c")
    base = wid * TPW
    pltpu.sync_copy(x_hbm.at[pl.ds(base, TPW)], rows_v)
    pltpu.sync_copy(destT_hbm.at[0, pl.ds(base, TPW)], idx_v)
    pltpu.async_copy(rows_v, xs_hbm.at[idx_v], sem).wait()
    pltpu.sync_copy(destT_hbm.at[1, pl.ds(base, TPW)], idx_v)
    pltpu.async_copy(rows_v, xs_hbm.at[idx_v], sem).wait()


@jax.jit
def _dispatch(x, destT):
    mesh = plsc.VectorSubcoreMesh(core_axis_name="c", subcore_axis_name="s")
    return pl.kernel(
        _dispatch_body,
        mesh=mesh,
        out_type=jax.ShapeDtypeStruct((P_PAD, H), jnp.float32),
        scratch_types=[
            pltpu.VMEM((TPW,), jnp.int32),
            pltpu.VMEM((TPW, H), jnp.float32),
            pltpu.SemaphoreType.DMA,
        ],
    )(x, destT)


def _ffn_body(be_ref, isf_ref, slt_ref, nxe_ref, hnx_ref,
              xs_ref, w1_hbm, b1_ref, w2_hbm, b2_ref, out_ref,
              w1s_ref, w2s_ref, sems):
    i = pl.program_id(0)
    first = isf_ref[i] == 1
    s = slt_ref[i]
    hn = hnx_ref[i] == 1
    ne = nxe_ref[i]

    @pl.when(i == 0)
    def _prime():
        pltpu.make_async_copy(w1_hbm.at[be_ref[0]], w1s_ref.at[0], sems.at[0]).start()
        pltpu.make_async_copy(w2_hbm.at[be_ref[0]], w2s_ref.at[0], sems.at[1]).start()

    @pl.when(first & (s == 0))
    def _wait0():
        pltpu.make_async_copy(w1_hbm.at[be_ref[i]], w1s_ref.at[0], sems.at[0]).wait()
        pltpu.make_async_copy(w2_hbm.at[be_ref[i]], w2s_ref.at[0], sems.at[1]).wait()

    @pl.when(first & (s == 1))
    def _wait1():
        pltpu.make_async_copy(w1_hbm.at[be_ref[i]], w1s_ref.at[1], sems.at[2]).wait()
        pltpu.make_async_copy(w2_hbm.at[be_ref[i]], w2s_ref.at[1], sems.at[3]).wait()

    @pl.when(first & hn & (s == 0))
    def _pref1():
        pltpu.make_async_copy(w1_hbm.at[ne], w1s_ref.at[1], sems.at[2]).start()
        pltpu.make_async_copy(w2_hbm.at[ne], w2s_ref.at[1], sems.at[3]).start()

    @pl.when(first & hn & (s == 1))
    def _pref0():
        pltpu.make_async_copy(w1_hbm.at[ne], w1s_ref.at[0], sems.at[0]).start()
        pltpu.make_async_copy(w2_hbm.at[ne], w2s_ref.at[0], sems.at[1]).start()

    xb = xs_ref[...]
    h1 = jnp.dot(xb, w1s_ref[s], preferred_element_type=jnp.float32)
    h1 = h1 + b1_ref[0]
    hmid = h1 * 0.5 * (1.0 + lax.erf(h1 * 0.7071067811865476))
    y = jnp.dot(hmid, w2s_ref[s], preferred_element_type=jnp.float32)
    out_ref[...] = y + b2_ref[0]


def _ffn(be, isf, slt, nxe, hnx, xs, W1, b1, W2, b2):
    grid_spec = pltpu.PrefetchScalarGridSpec(
        num_scalar_prefetch=5,
        grid=(NBLK,),
        in_specs=[
            pl.BlockSpec((BM, H), lambda i, *_: (i, 0)),
            pl.BlockSpec(memory_space=pl.ANY),
            pl.BlockSpec((1, 1, FF), lambda i, be, *_: (be[i], 0, 0)),
            pl.BlockSpec(memory_space=pl.ANY),
            pl.BlockSpec((1, 1, H), lambda i, be, *_: (be[i], 0, 0)),
        ],
        out_specs=pl.BlockSpec((BM, H), lambda i, *_: (i, 0)),
        scratch_shapes=[
            pltpu.VMEM((2, H, FF), jnp.float32),
            pltpu.VMEM((2, FF, H), jnp.float32),
            pltpu.SemaphoreType.DMA((4,)),
        ],
    )
    return pl.pallas_call(
        _ffn_body,
        grid_spec=grid_spec,
        out_shape=jax.ShapeDtypeStruct((P_PAD, H), jnp.float32),
    )(be, isf, slt, nxe, hnx,
      xs, W1, b1.reshape(E, 1, FF), W2, b2.reshape(E, 1, H))


def _combine_body(ys_hbm, destT_hbm, wexp_hbm, out_hbm,
                  idx_v, w_v, rows0_v, rows1_v, sem):
    wid = lax.axis_index("s") * 2 + lax.axis_index("c")
    base = wid * TPW
    pltpu.sync_copy(destT_hbm.at[0, pl.ds(base, TPW)], idx_v)
    pltpu.async_copy(ys_hbm.at[idx_v], rows0_v, sem).wait()
    pltpu.sync_copy(destT_hbm.at[1, pl.ds(base, TPW)], idx_v)
    pltpu.async_copy(ys_hbm.at[idx_v], rows1_v, sem).wait()
    pltpu.sync_copy(wexp_hbm.at[pl.ds(base, TPW)], w_v)

    def tok(j, _):
        wv0 = w_v[j, pl.ds(0, 16)]
        wv1 = w_v[j, pl.ds(16, 16)]
        for c in range(H // 16):
            sl = pl.ds(c * 16, 16)
            v = rows0_v[j, sl] * wv0 + rows1_v[j, sl] * wv1
            rows0_v[j, sl] = v
        return 0

    lax.fori_loop(0, TPW, tok, 0)
    pltpu.sync_copy(rows0_v, out_hbm.at[pl.ds(base, TPW)])


@jax.jit
def _combine(ys, destT, wexp):
    mesh = plsc.VectorSubcoreMesh(core_axis_name="c", subcore_axis_name="s")
    return pl.kernel(
        _combine_body,
        mesh=mesh,
        out_type=jax.ShapeDtypeStruct((T, H), jnp.float32),
        scratch_types=[
            pltpu.VMEM((TPW,), jnp.int32),
            pltpu.VMEM((TPW, 32), jnp.float32),
            pltpu.VMEM((TPW, H), jnp.float32),
            pltpu.VMEM((TPW, H), jnp.float32),
            pltpu.SemaphoreType.DMA,
        ],
    )(ys, destT, wexp)


def kernel(hidden_states, Wr, W1, b1, W2, b2):
    x = hidden_states.reshape(T, H)
    wr_pad = jnp.zeros((H, LANES), jnp.float32).at[:, :E].set(Wr)
    sel, ew, wexp, dest, counts, loss = _router(x, wr_pad)
    cnt = counts[0, :E].astype(jnp.int32)
    pad_count = ((cnt + (BM - 1)) // BM) * BM
    pad_start = jnp.cumsum(pad_count) - pad_count
    blk = jnp.arange(NBLK, dtype=jnp.int32) * BM
    block_expert = jnp.sum(blk[:, None] >= pad_start[None, :],
                           axis=1).astype(jnp.int32) - 1
    block_expert = jnp.clip(block_expert, 0, E - 1)
    be = block_expert
    isf = jnp.concatenate([jnp.ones((1,), jnp.int32),
                           (be[1:] != be[:-1]).astype(jnp.int32)])
    seg = jnp.cumsum(isf) - 1
    slt = (seg % 2).astype(jnp.int32)
    seg_expert = jnp.zeros((NBLK,), jnp.int32).at[seg].set(be)
    nxe = seg_expert[jnp.minimum(seg + 1, NBLK - 1)]
    hnx = (seg < seg[-1]).astype(jnp.int32)
    destT = dest.T
    xs = _dispatch(x, destT)
    ys = _ffn(be, isf, slt, nxe, hnx, xs, W1, b1, W2, b2)
    out = _combine(ys, destT, wexp)
    return (out.reshape(B, S, H), sel, ew, loss[0, 0])
